# baseline (speedup 1.0000x reference)
import functools

import jax
import jax.numpy as jnp
from jax import lax
from jax.experimental import pallas as pl
from jax.experimental.pallas import tpu as pltpu
from jax.experimental.pallas import tpu_sc as plsc

_E = 160000
_D = 256
_L = 16
_C = 112
_NW = 32
_P = 23
_CT = 2 * _P
_EPAD = _NW * _CT * _C


def _sc_body(z_hbm, src_hbm, dst_hbm, out_hbm, sidx, didx,
             srows_a, drows_a, srows_b, drows_b, outv_a, outv_b, tbuf,
             sem_a, sem_b, sem_oa, sem_ob):
    w = lax.axis_index("s") * 2 + lax.axis_index("c")
    pltpu.sync_copy(src_hbm.at[w], sidx)
    pltpu.sync_copy(dst_hbm.at[w], didx)

    def _gather(c, srows, drows, sem):
        pltpu.async_copy(z_hbm.at[sidx.at[c]], srows, sem)
        pltpu.async_copy(z_hbm.at[didx.at[c]], drows, sem)

    def _drain(srows, drows, sem):
        pltpu.make_async_copy(z_hbm.at[pl.ds(0, _C)], srows, sem).wait()
        pltpu.make_async_copy(z_hbm.at[pl.ds(0, _C)], drows, sem).wait()

    lane = lax.iota(jnp.int32, _L)

    def _compute(srows, drows, outv):
        @pl.loop(0, _C // _L)
        def _group(g):
            for j in range(_L):
                e = g * _L + j
                acc0 = srows[e, pl.ds(0, _L)] * drows[e, pl.ds(0, _L)]
                acc1 = srows[e, pl.ds(_L, _L)] * drows[e, pl.ds(_L, _L)]
                for i in range(2, _D // _L, 2):
                    acc0 = acc0 + (srows[e, pl.ds(i * _L, _L)]
                                   * drows[e, pl.ds(i * _L, _L)])
                    acc1 = acc1 + (srows[e, pl.ds((i + 1) * _L, _L)]
                                   * drows[e, pl.ds((i + 1) * _L, _L)])
                tbuf[pl.ds(j * _L, _L)] = acc0 + acc1
            score = plsc.load_gather(tbuf, [lane * _L])
            for i in range(1, _L):
                score = score + plsc.load_gather(tbuf, [lane * _L + i])
            outv[pl.ds(g * _L, _L)] = 1.0 / (1.0 + jnp.exp(-score))

    def _out_issue(c, outv, sem):
        pltpu.async_copy(outv, out_hbm.at[pl.ds((w * _CT + c) * _C, _C)], sem)

    def _out_drain(outv, sem):
        pltpu.make_async_copy(outv, out_hbm.at[pl.ds(0, _C)], sem).wait()

    _gather(0, srows_a, drows_a, sem_a)

    @pl.loop(0, _P)
    def _pair(p):
        c0 = 2 * p
        _gather(c0 + 1, srows_b, drows_b, sem_b)
        _drain(srows_a, drows_a, sem_a)

        @pl.when(p > 0)
        def _da():
            _out_drain(outv_a, sem_oa)

        _compute(srows_a, drows_a, outv_a)
        _out_issue(c0, outv_a, sem_oa)

        @pl.when(p < _P - 1)
        def _issue_a():
            _gather(c0 + 2, srows_a, drows_a, sem_a)

        _drain(srows_b, drows_b, sem_b)

        @pl.when(p > 0)
        def _db():
            _out_drain(outv_b, sem_ob)

        _compute(srows_b, drows_b, outv_b)
        _out_issue(c0 + 1, outv_b, sem_ob)

    _out_drain(outv_a, sem_oa)
    _out_drain(outv_b, sem_ob)


def kernel(z, edge_index):
    src = edge_index[0].astype(jnp.int32)
    dst = edge_index[1].astype(jnp.int32)
    pad = jnp.arange(_EPAD - _E, dtype=jnp.int32) % 10000
    src = jnp.concatenate([src, pad]).reshape(_NW, _CT, _C)
    dst = jnp.concatenate([dst, pad]).reshape(_NW, _CT, _C)
    mesh = plsc.VectorSubcoreMesh(core_axis_name="c", subcore_axis_name="s")
    run = functools.partial(
        pl.kernel,
        mesh=mesh,
        compiler_params=pltpu.CompilerParams(needs_layout_passes=False),
        out_type=jax.ShapeDtypeStruct((_EPAD,), jnp.float32),
        scratch_types=[
            pltpu.VMEM((_CT, _C), jnp.int32),
            pltpu.VMEM((_CT, _C), jnp.int32),
            pltpu.VMEM((_C, _D), jnp.float32),
            pltpu.VMEM((_C, _D), jnp.float32),
            pltpu.VMEM((_C, _D), jnp.float32),
            pltpu.VMEM((_C, _D), jnp.float32),
            pltpu.VMEM((_C,), jnp.float32),
            pltpu.VMEM((_C,), jnp.float32),
            pltpu.VMEM((_L * _L,), jnp.float32),
            pltpu.SemaphoreType.DMA,
            pltpu.SemaphoreType.DMA,
            pltpu.SemaphoreType.DMA,
            pltpu.SemaphoreType.DMA,
        ],
    )(_sc_body)
    return run(z, src, dst)[:_E]


# speedup vs baseline: 1.0409x; 1.0409x over previous
import functools

import jax
import jax.numpy as jnp
from jax import lax
from jax.experimental import pallas as pl
from jax.experimental.pallas import tpu as pltpu
from jax.experimental.pallas import tpu_sc as plsc

_E = 160000
_D = 256
_L = 16
_C = 112
_NW = 32
_P = 23
_CT = 2 * _P
_EPAD = _NW * _CT * _C


def _sc_body(z_hbm, src_hbm, dst_hbm, out_hbm, sidx, didx,
             srows_a, drows_a, srows_b, drows_b, outv_a, outv_b, tbuf,
             sem_a, sem_b, sem_oa, sem_ob):
    w = lax.axis_index("s") * 2 + lax.axis_index("c")
    pltpu.sync_copy(src_hbm.at[w], sidx)
    pltpu.sync_copy(dst_hbm.at[w], didx)

    def _gather(c, srows, drows, sem):
        pltpu.async_copy(z_hbm.at[sidx.at[c]], srows, sem)
        pltpu.async_copy(z_hbm.at[didx.at[c]], drows, sem)

    def _drain(srows, drows, sem):
        pltpu.make_async_copy(z_hbm.at[pl.ds(0, _C)], srows, sem).wait()
        pltpu.make_async_copy(z_hbm.at[pl.ds(0, _C)], drows, sem).wait()

    lane = lax.iota(jnp.int32, _L)

    def _compute(srows, drows, outv):
        @pl.loop(0, _C // _L)
        def _group(g):
            for j in range(_L):
                e = g * _L + j
                acc = srows[e, pl.ds(0, _L)] * drows[e, pl.ds(0, _L)]
                for i in range(1, _D // _L):
                    acc = acc + (srows[e, pl.ds(i * _L, _L)]
                                 * drows[e, pl.ds(i * _L, _L)])
                tbuf[pl.ds(j * _L, _L)] = acc
            score = plsc.load_gather(tbuf, [lane * _L])
            for i in range(1, _L):
                score = score + plsc.load_gather(tbuf, [lane * _L + i])
            outv[pl.ds(g * _L, _L)] = 1.0 / (1.0 + jnp.exp(-score))

    def _out_issue(c, outv, sem):
        pltpu.async_copy(outv, out_hbm.at[pl.ds((w * _CT + c) * _C, _C)], sem)

    def _out_drain(outv, sem):
        pltpu.make_async_copy(outv, out_hbm.at[pl.ds(0, _C)], sem).wait()

    _gather(0, srows_a, drows_a, sem_a)

    @pl.loop(0, _P)
    def _pair(p):
        c0 = 2 * p
        _gather(c0 + 1, srows_b, drows_b, sem_b)
        _drain(srows_a, drows_a, sem_a)

        @pl.when(p > 0)
        def _da():
            _out_drain(outv_a, sem_oa)

        _compute(srows_a, drows_a, outv_a)
        _out_issue(c0, outv_a, sem_oa)

        @pl.when(p < _P - 1)
        def _issue_a():
            _gather(c0 + 2, srows_a, drows_a, sem_a)

        _drain(srows_b, drows_b, sem_b)

        @pl.when(p > 0)
        def _db():
            _out_drain(outv_b, sem_ob)

        _compute(srows_b, drows_b, outv_b)
        _out_issue(c0 + 1, outv_b, sem_ob)

    _out_drain(outv_a, sem_oa)
    _out_drain(outv_b, sem_ob)


def kernel(z, edge_index):
    src = edge_index[0].astype(jnp.int32)
    dst = edge_index[1].astype(jnp.int32)
    pad = jnp.arange(_EPAD - _E, dtype=jnp.int32) % 10000
    src = jnp.concatenate([src, pad]).reshape(_NW, _CT, _C)
    dst = jnp.concatenate([dst, pad]).reshape(_NW, _CT, _C)
    mesh = plsc.VectorSubcoreMesh(core_axis_name="c", subcore_axis_name="s")
    run = functools.partial(
        pl.kernel,
        mesh=mesh,
        compiler_params=pltpu.CompilerParams(needs_layout_passes=False),
        out_type=jax.ShapeDtypeStruct((_EPAD,), jnp.float32),
        scratch_types=[
            pltpu.VMEM((_CT, _C), jnp.int32),
            pltpu.VMEM((_CT, _C), jnp.int32),
            pltpu.VMEM((_C, _D), jnp.float32),
            pltpu.VMEM((_C, _D), jnp.float32),
            pltpu.VMEM((_C, _D), jnp.float32),
            pltpu.VMEM((_C, _D), jnp.float32),
            pltpu.VMEM((_C,), jnp.float32),
            pltpu.VMEM((_C,), jnp.float32),
            pltpu.VMEM((_L * _L,), jnp.float32),
            pltpu.SemaphoreType.DMA,
            pltpu.SemaphoreType.DMA,
            pltpu.SemaphoreType.DMA,
            pltpu.SemaphoreType.DMA,
        ],
    )(_sc_body)
    return run(z, src, dst)[:_E]
